# Initial kernel scaffold; baseline (speedup 1.0000x reference)
import jax
import jax.numpy as jnp
from jax.experimental import pallas as pl


def kernel(x, train_x, train_y, w):
    raise NotImplementedError("write your pallas kernel here")


# speedup vs baseline: 3.4229x; 3.4229x over previous
import jax
import jax.numpy as jnp
from jax.experimental import pallas as pl
from jax.experimental.pallas import tpu as pltpu

N_QUERY = 4096
N_TRAIN = 16384
N_FEAT = 128
N_OUT = 16
TOP_K = 16
Q_BLOCK = 256
N_SELECT_ITERS = TOP_K + 1


def _idw_block_kernel(x_ref, tst_ref, ty_ref, w_ref, out_ref, dist_ref, y2_ref):
    s2 = jnp.exp(-2.0 * w_ref[0, :])

    @pl.when(pl.program_id(0) == 0)
    def _():
        tst = tst_ref[...]
        y2_ref[...] = jnp.sum(tst * tst * s2[:, None], axis=0, keepdims=True)

    x = x_ref[...]
    xs = x * s2[None, :]
    x2 = jnp.sum(x * xs, axis=1, keepdims=True)
    cross = jnp.dot(xs, tst_ref[...],
                    preferred_element_type=jnp.float32)
    sq = jnp.maximum(x2 + y2_ref[...] - 2.0 * cross, 0.0)
    dist_ref[...] = 1.0 / jnp.sqrt(sq + 1e-6)

    d = dist_ref[...]

    m = jnp.full((Q_BLOCK, 1), jnp.inf, jnp.float32)
    cum = jnp.zeros((Q_BLOCK, 1), jnp.float32)
    thr = jnp.zeros((Q_BLOCK, 1), jnp.float32)
    found = jnp.zeros((Q_BLOCK, 1), jnp.float32)
    for _ in range(N_SELECT_ITERS):
        cnt = jnp.sum(jnp.where(d == m, 1.0, 0.0), axis=1, keepdims=True)
        cum = cum + cnt
        newly = (1.0 - found) * jnp.where(cum >= float(TOP_K), 1.0, 0.0)
        thr = jnp.where(newly > 0.0, m, thr)
        found = jnp.maximum(found, newly)
        m = jnp.max(jnp.where(d < m, d, -jnp.inf), axis=1, keepdims=True)

    wts = jnp.where(d >= thr, d, 0.0)
    denom = jnp.sum(wts, axis=1, keepdims=True)
    num = jnp.dot(wts, ty_ref[...],
                  preferred_element_type=jnp.float32)
    out_ref[...] = num / denom


def kernel(x, train_x, train_y, w):
    tst = train_x.T
    w2d = w.reshape(1, N_FEAT)
    return pl.pallas_call(
        _idw_block_kernel,
        grid=(N_QUERY // Q_BLOCK,),
        in_specs=[
            pl.BlockSpec((Q_BLOCK, N_FEAT), lambda i: (i, 0)),
            pl.BlockSpec((N_FEAT, N_TRAIN), lambda i: (0, 0)),
            pl.BlockSpec((N_TRAIN, N_OUT), lambda i: (0, 0)),
            pl.BlockSpec((1, N_FEAT), lambda i: (0, 0)),
        ],
        out_specs=pl.BlockSpec((Q_BLOCK, N_OUT), lambda i: (i, 0)),
        out_shape=jax.ShapeDtypeStruct((N_QUERY, N_OUT), jnp.float32),
        scratch_shapes=[
            pltpu.VMEM((Q_BLOCK, N_TRAIN), jnp.float32),
            pltpu.VMEM((1, N_TRAIN), jnp.float32),
        ],
        compiler_params=pltpu.CompilerParams(
            dimension_semantics=("arbitrary",)),
    )(x, tst, train_y, w2d)


# speedup vs baseline: 9.9092x; 2.8950x over previous
import jax
import jax.numpy as jnp
from jax.experimental import pallas as pl
from jax.experimental.pallas import tpu as pltpu

N_QUERY = 4096
N_TRAIN = 16384
N_FEAT = 128
N_OUT = 16
TOP_K = 16
Q_BLOCK = 256


def _bitonic_merge(lst):
    n = len(lst)
    if n == 1:
        return lst
    h = n // 2
    hi = [jnp.maximum(lst[i], lst[i + h]) for i in range(h)]
    lo = [jnp.minimum(lst[i], lst[i + h]) for i in range(h)]
    return _bitonic_merge(hi) + _bitonic_merge(lo)


def _idw_block_kernel(x_ref, tst_ref, ty_ref, w_ref, out_ref, dist_ref, y2_ref):
    s2 = jnp.exp(-2.0 * w_ref[0, :])

    @pl.when(pl.program_id(0) == 0)
    def _():
        tst = tst_ref[...]
        y2_ref[...] = jnp.sum(tst * tst * s2[:, None], axis=0, keepdims=True)

    x = x_ref[...]
    xs = x * s2[None, :]
    x2 = jnp.sum(x * xs, axis=1, keepdims=True)
    cross = jnp.dot(xs, tst_ref[...],
                    preferred_element_type=jnp.float32)
    sq = jnp.maximum(x2 + y2_ref[...] - 2.0 * cross, 0.0)
    dist_ref[...] = 1.0 / jnp.sqrt(sq + 1e-6)

    d = dist_ref[...]

    lst = [d]
    while len(lst) < TOP_K or lst[0].shape[1] > 1:
        k = len(lst)
        half = lst[0].shape[1] // 2
        a = [t[:, :half] for t in lst]
        b = [t[:, half:] for t in lst]
        rev = b[::-1]
        hi = [jnp.maximum(a[i], rev[i]) for i in range(k)]
        if k < TOP_K:
            lo = [jnp.minimum(a[i], rev[i]) for i in range(k)]
            lst = _bitonic_merge(hi) + _bitonic_merge(lo)
        else:
            lst = _bitonic_merge(hi)
    thr = lst[TOP_K - 1]

    wts = jnp.where(d >= thr, d, 0.0)
    denom = jnp.sum(wts, axis=1, keepdims=True)
    num = jnp.dot(wts, ty_ref[...],
                  preferred_element_type=jnp.float32)
    out_ref[...] = num / denom


def kernel(x, train_x, train_y, w):
    tst = train_x.T
    w2d = w.reshape(1, N_FEAT)
    return pl.pallas_call(
        _idw_block_kernel,
        grid=(N_QUERY // Q_BLOCK,),
        in_specs=[
            pl.BlockSpec((Q_BLOCK, N_FEAT), lambda i: (i, 0)),
            pl.BlockSpec((N_FEAT, N_TRAIN), lambda i: (0, 0)),
            pl.BlockSpec((N_TRAIN, N_OUT), lambda i: (0, 0)),
            pl.BlockSpec((1, N_FEAT), lambda i: (0, 0)),
        ],
        out_specs=pl.BlockSpec((Q_BLOCK, N_OUT), lambda i: (i, 0)),
        out_shape=jax.ShapeDtypeStruct((N_QUERY, N_OUT), jnp.float32),
        scratch_shapes=[
            pltpu.VMEM((Q_BLOCK, N_TRAIN), jnp.float32),
            pltpu.VMEM((1, N_TRAIN), jnp.float32),
        ],
        compiler_params=pltpu.CompilerParams(
            dimension_semantics=("arbitrary",)),
    )(x, tst, train_y, w2d)
